# baseline (speedup 1.0000x reference)
import functools

import jax
import jax.numpy as jnp
from jax import lax
from jax.experimental import pallas as pl
from jax.experimental.pallas import tpu as pltpu
from jax.experimental.pallas import tpu_sc as plsc

_BATCH = 16384
_Z_DIM = 64
_NUM_WORKERS = 32
_B_PER_W = _BATCH // _NUM_WORKERS
_IDX_ROW = 128
_N_CHUNKS = _B_PER_W // _IDX_ROW


def kernel(u, embed_mean, embed_log_var):
    idx = u.astype(jnp.int32).reshape(_BATCH // _IDX_ROW, _IDX_ROW)
    out_sds = jax.ShapeDtypeStruct((_BATCH, _Z_DIM), embed_mean.dtype)
    mesh = plsc.VectorSubcoreMesh(core_axis_name="c", subcore_axis_name="s")

    @jax.jit
    @functools.partial(
        pl.kernel,
        out_type=(out_sds, out_sds),
        mesh=mesh,
        compiler_params=pltpu.CompilerParams(
            skip_device_barrier=True, use_tc_tiling_on_sc=False),
        scratch_types=[
            pltpu.VMEM((_N_CHUNKS, _IDX_ROW), jnp.int32),
            pltpu.VMEM((_B_PER_W, _Z_DIM), jnp.float32),
            pltpu.VMEM((_B_PER_W, _Z_DIM), jnp.float32),
            pltpu.SemaphoreType.DMA,
            [pltpu.SemaphoreType.DMA for _ in range(_N_CHUNKS)],
            [pltpu.SemaphoreType.DMA for _ in range(_N_CHUNKS)],
            pltpu.SemaphoreType.DMA,
            pltpu.SemaphoreType.DMA,
        ],
    )
    def _gather(mean_hbm, logvar_hbm, idx_hbm, om_hbm, ov_hbm,
                idx_v, mrows, vrows, sem_i, sems_m, sems_v, sem_wm, sem_wv):
        wid = lax.axis_index("s") * 2 + lax.axis_index("c")
        base = wid * _B_PER_W
        pltpu.async_copy(
            idx_hbm.at[pl.ds(wid * _N_CHUNKS, _N_CHUNKS)], idx_v, sem_i
        ).wait()

        mcopies = []
        vcopies = []
        for j in range(_N_CHUNKS):
            rows = pl.ds(j * _IDX_ROW, _IDX_ROW)
            mcopies.append(pltpu.async_copy(
                mean_hbm.at[idx_v.at[j]], mrows.at[rows], sems_m[j]))
            vcopies.append(pltpu.async_copy(
                logvar_hbm.at[idx_v.at[j]], vrows.at[rows], sems_v[j]))

        wcopies = []
        for j in range(_N_CHUNKS):
            rows = pl.ds(j * _IDX_ROW, _IDX_ROW)
            out_rows = pl.ds(base + j * _IDX_ROW, _IDX_ROW)
            mcopies[j].wait()
            wcopies.append(pltpu.async_copy(
                mrows.at[rows], om_hbm.at[out_rows], sem_wm))
            vcopies[j].wait()
            wcopies.append(pltpu.async_copy(
                vrows.at[rows], ov_hbm.at[out_rows], sem_wv))
        for c in wcopies:
            c.wait()

    return _gather(embed_mean, embed_log_var, idx)
